# Initial kernel scaffold; baseline (speedup 1.0000x reference)
import jax
import jax.numpy as jnp
from jax.experimental import pallas as pl


def kernel(masked_layer):
    raise NotImplementedError("write your pallas kernel here")


# speedup vs baseline: 1.8041x; 1.8041x over previous
import functools

import jax
import jax.numpy as jnp
from jax import lax
from jax.experimental import pallas as pl
from jax.experimental.pallas import tpu as pltpu
from jax.experimental.pallas import tpu_sc as plsc

K = 64
B = 32
N = 4096
D = 128
LANES = 16
DGROUPS = D // LANES
NUM_TASKS = B * DGROUPS
NB1 = 2048
NB2 = 2048
NB3 = 1024


def _to_key(x):
    i = plsc.bitcast(x, jnp.int32)
    return jnp.where(i >= 0, i, i ^ jnp.int32(0x7FFFFFFF))


def _from_key(key):
    i = jnp.where(key >= 0, key, key ^ jnp.int32(0x7FFFFFFF))
    return plsc.bitcast(i, jnp.float32)


def _walk(hist_ref, nbuckets, acc0):
    zero16 = jnp.zeros((LANES,), jnp.int32)

    def cond(carry):
        j, acc, _, _ = carry
        return (j >= 0) & (jnp.min(acc) < K)

    def body(carry):
        j, acc, bsel, above = carry
        h = hist_ref[j]
        nacc = acc + h
        newly = (acc < K) & (nacc >= K)
        bsel = jnp.where(newly, j, bsel)
        above = jnp.where(newly, acc, above)
        return j - 1, nacc, bsel, above

    _, _, bsel, above = lax.while_loop(
        cond, body, (jnp.int32(nbuckets - 1), acc0, zero16, zero16))
    return bsel, above


def _clear_hist(hist_ref, nbuckets):
    zero16 = jnp.zeros((LANES,), jnp.int32)

    def body(j, _):
        hist_ref[j] = zero16
        return 0

    lax.fori_loop(0, nbuckets, body, 0)


def _bitonic_sort_desc(g_ref):
    for k in range(6):
        for j in range(k, -1, -1):
            s = 1 << j

            def body(i, _, j=j, s=s, k=k):
                p = ((i >> j) << (j + 1)) | (i & (s - 1))
                q = p | s
                a = g_ref[p]
                b = g_ref[q]
                mx = jnp.maximum(a, b)
                mn = jnp.minimum(a, b)
                desc = ((p >> (k + 1)) & 1) == 0
                g_ref[p] = jnp.where(desc, mx, mn)
                g_ref[q] = jnp.where(desc, mn, mx)
                return 0

            lax.fori_loop(0, 32, body, 0)


def _task_body(x_hbm, o_hbm, data_v, hist_v, g_v, out_v, b, dg):
    lane = lax.iota(jnp.int32, LANES)
    ones = jnp.ones((LANES,), jnp.int32)
    zero16 = jnp.zeros((LANES,), jnp.int32)
    d0 = dg * LANES

    pltpu.sync_copy(x_hbm.at[b, :, pl.ds(d0, LANES)], data_v)

    _clear_hist(hist_v, NB1)

    def p1(n, _):
        key = _to_key(data_v[n])
        data_v[n] = plsc.bitcast(key, jnp.float32)
        b1 = (key >> 21) + 1024
        plsc.addupdate_scatter(hist_v, [b1, lane], ones)
        return 0

    lax.fori_loop(0, N, p1, 0)
    bsel1, c1 = _walk(hist_v, NB1, zero16)

    _clear_hist(hist_v, NB2)

    def p2(n, _):
        key = plsc.bitcast(data_v[n], jnp.int32)
        m = ((key >> 21) + 1024) == bsel1
        b2 = (key >> 10) & 0x7FF
        plsc.addupdate_scatter(hist_v, [b2, lane], ones, mask=m)
        return 0

    lax.fori_loop(0, N, p2, 0)
    bsel2, c2 = _walk(hist_v, NB2, c1)

    prefix22 = ((bsel1 - 1024) << 11) | bsel2
    _clear_hist(hist_v, NB3)

    def p3(n, _):
        key = plsc.bitcast(data_v[n], jnp.int32)
        m = (key >> 10) == prefix22
        b3 = key & 0x3FF
        plsc.addupdate_scatter(hist_v, [b3, lane], ones, mask=m)
        return 0

    lax.fori_loop(0, N, p3, 0)
    bsel3, _ = _walk(hist_v, NB3, c2)

    t_key = (prefix22 << 10) | bsel3

    def fill(kk, _):
        g_v[kk] = t_key
        return 0

    lax.fori_loop(0, K, fill, 0)

    def p4(n, cnt):
        key = plsc.bitcast(data_v[n], jnp.int32)
        m = key > t_key
        plsc.store_scatter(g_v, [cnt, lane], key, mask=m)
        return cnt + jnp.where(m, 1, 0)

    lax.fori_loop(0, N, p4, zero16)

    _bitonic_sort_desc(g_v)

    def emit(kk, _):
        x = _from_key(g_v[kk])
        plsc.store_scatter(out_v, [lane * K + kk], x)
        return 0

    lax.fori_loop(0, K, emit, 0)
    pltpu.sync_copy(out_v, o_hbm.at[b, pl.ds(dg * K * LANES, K * LANES)])


def _sc_topk(x):
    nc, ns = 2, 16
    nw = nc * ns
    tasks_per_w = NUM_TASKS // nw
    mesh = plsc.VectorSubcoreMesh(
        core_axis_name="c", subcore_axis_name="s", num_cores=nc, num_subcores=ns)

    @functools.partial(
        pl.kernel,
        out_type=jax.ShapeDtypeStruct((B, K * D), jnp.float32),
        mesh=mesh,
        scratch_types=[
            pltpu.VMEM((N, LANES), jnp.float32),
            pltpu.VMEM((NB1, LANES), jnp.int32),
            pltpu.VMEM((K, LANES), jnp.int32),
            pltpu.VMEM((K * LANES,), jnp.float32),
        ],
        compiler_params=pltpu.CompilerParams(
            use_tc_tiling_on_sc=False, needs_layout_passes=False),
    )
    def kern(x_hbm, o_hbm, data_v, hist_v, g_v, out_v):
        wid = lax.axis_index("s") * nc + lax.axis_index("c")

        def task(t, _):
            tid = wid * tasks_per_w + t
            bb = tid // DGROUPS
            dg = tid % DGROUPS
            _task_body(x_hbm, o_hbm, data_v, hist_v, g_v, out_v, bb, dg)
            return 0

        lax.fori_loop(0, tasks_per_w, task, 0)

    return kern(x)


def kernel(masked_layer):
    return _sc_topk(masked_layer)


# speedup vs baseline: 10.8952x; 6.0393x over previous
import functools

import jax
import jax.numpy as jnp
from jax import lax
from jax.experimental import pallas as pl
from jax.experimental.pallas import tpu as pltpu
from jax.experimental.pallas import tpu_sc as plsc

K = 64
B = 32
N = 4096
D = 128
LANES = 16
DGROUPS = D // LANES
NUM_TASKS = B * DGROUPS
NB1 = 1024
NBB = 256
NBC = 128


def _to_key(x):
    i = plsc.bitcast(x, jnp.int32)
    return jnp.where(i >= 0, i, i ^ jnp.int32(0x7FFFFFFF))


def _from_key(key):
    i = jnp.where(key >= 0, key, key ^ jnp.int32(0x7FFFFFFF))
    return plsc.bitcast(i, jnp.float32)


def _walk(hist_ref, start, acc0):
    zero16 = jnp.zeros((LANES,), jnp.int32)

    def scond(j):
        return (j >= 0) & (jnp.max(hist_ref[j]) == 0)

    j0 = lax.while_loop(scond, lambda j: j - 1, jnp.int32(start))

    def cond(carry):
        j, acc, _, _ = carry
        return (j >= 0) & (jnp.min(acc) < K)

    def body(carry):
        j, acc, bsel, above = carry
        h = hist_ref[j]
        nacc = acc + h
        newly = (acc < K) & (nacc >= K)
        bsel = jnp.where(newly, j, bsel)
        above = jnp.where(newly, acc, above)
        return j - 1, nacc, bsel, above

    _, _, bsel, above = lax.while_loop(cond, body, (j0, acc0, zero16, zero16))
    return bsel, above


def _clear(hist_ref, nbuckets):
    zero16 = jnp.zeros((LANES,), jnp.int32)

    @plsc.parallel_loop(0, nbuckets, unroll=8)
    def clr(j):
        hist_ref[j] = zero16


def _bitonic_sort_desc(g_ref):
    for k in range(6):
        for j in range(k, -1, -1):
            s = 1 << j

            @plsc.parallel_loop(0, 32, unroll=4)
            def cx(i, j=j, s=s, k=k):
                p = ((i >> j) << (j + 1)) | (i & (s - 1))
                q = p | s
                a = g_ref[p]
                b = g_ref[q]
                mx = jnp.maximum(a, b)
                mn = jnp.minimum(a, b)
                desc = ((p >> (k + 1)) & 1) == 0
                g_ref[p] = jnp.where(desc, mx, mn)
                g_ref[q] = jnp.where(desc, mn, mx)


def _task_body(x_hbm, o_hbm, data_v, hist_v, histb_v, g_v, out_v, b, dg):
    lane = lax.iota(jnp.int32, LANES)
    ones = jnp.ones((LANES,), jnp.int32)
    zero16 = jnp.zeros((LANES,), jnp.int32)
    d0 = dg * LANES

    pltpu.sync_copy(x_hbm.at[b, :, pl.ds(d0, LANES)], data_v)

    _clear(hist_v, NB1)

    @plsc.parallel_loop(0, N, unroll=4)
    def p1(n):
        key = _to_key(data_v[n])
        data_v[n] = plsc.bitcast(key, jnp.float32)
        plsc.addupdate_scatter(hist_v, [(key >> 22) + 512, lane], ones)

    bsel1, c1 = _walk(hist_v, NB1 - 1, zero16)
    top1 = bsel1 - 512

    @plsc.parallel_loop(0, N, unroll=4, carry=(zero16, zero16))
    def p2(n, carry):
        ch, ce = carry
        key = plsc.bitcast(data_v[n], jnp.int32)
        b1 = key >> 22
        m_hi = b1 > top1
        m_eq = b1 == top1
        plsc.store_scatter(g_v, [ch, lane], key, mask=m_hi)
        plsc.store_scatter(data_v, [ce, lane], plsc.bitcast(key, jnp.float32),
                           mask=m_eq)
        return ch + jnp.where(m_hi, 1, 0), ce + jnp.where(m_eq, 1, 0)

    c_hi, c_eq = p2
    ncand = jnp.max(c_eq)

    _clear(histb_v, NBB)

    @plsc.parallel_loop(0, ncand, unroll=2)
    def pb(n):
        key = plsc.bitcast(data_v[n], jnp.int32)
        m = c_eq > n
        plsc.addupdate_scatter(histb_v, [(key >> 14) & 0xFF, lane], ones,
                               mask=m)

    bselB, cB = _walk(histb_v, NBB - 1, c1)

    _clear(histb_v, NBC)

    @plsc.parallel_loop(0, ncand, unroll=2)
    def pc(n):
        key = plsc.bitcast(data_v[n], jnp.int32)
        m = (c_eq > n) & (((key >> 14) & 0xFF) == bselB)
        plsc.addupdate_scatter(histb_v, [(key >> 7) & 0x7F, lane], ones,
                               mask=m)

    bselC, cC = _walk(histb_v, NBC - 1, cB)

    preBC = (bselB << 7) | bselC
    _clear(histb_v, NBC)

    @plsc.parallel_loop(0, ncand, unroll=2)
    def pd(n):
        key = plsc.bitcast(data_v[n], jnp.int32)
        m = (c_eq > n) & (((key >> 7) & 0x7FFF) == preBC)
        plsc.addupdate_scatter(histb_v, [key & 0x7F, lane], ones, mask=m)

    bselD, _ = _walk(histb_v, NBC - 1, cC)

    t_key = (top1 << 22) | (bselB << 14) | (bselC << 7) | bselD

    @plsc.parallel_loop(0, ncand, unroll=2, carry=c_hi)
    def p4(n, ct):
        key = plsc.bitcast(data_v[n], jnp.int32)
        m = (c_eq > n) & (key > t_key)
        plsc.store_scatter(g_v, [ct, lane], key, mask=m)
        return ct + jnp.where(m, 1, 0)

    total = p4

    @plsc.parallel_loop(0, K, unroll=4)
    def fill(kk):
        cur = g_v[kk]
        g_v[kk] = jnp.where(total <= kk, t_key, cur)

    _bitonic_sort_desc(g_v)

    lane64 = lane * K

    @plsc.parallel_loop(0, K, unroll=4)
    def emit(kk):
        x = _from_key(g_v[kk])
        plsc.store_scatter(out_v, [lane64 + kk], x)

    pltpu.sync_copy(out_v, o_hbm.at[b, pl.ds(dg * K * LANES, K * LANES)])


def _sc_topk(x):
    nc, ns = 2, 16
    nw = nc * ns
    tasks_per_w = NUM_TASKS // nw
    mesh = plsc.VectorSubcoreMesh(
        core_axis_name="c", subcore_axis_name="s", num_cores=nc, num_subcores=ns)

    @functools.partial(
        pl.kernel,
        out_type=jax.ShapeDtypeStruct((B, K * D), jnp.float32),
        mesh=mesh,
        scratch_types=[
            pltpu.VMEM((N, LANES), jnp.float32),
            pltpu.VMEM((NB1, LANES), jnp.int32),
            pltpu.VMEM((NBB, LANES), jnp.int32),
            pltpu.VMEM((K, LANES), jnp.int32),
            pltpu.VMEM((K * LANES,), jnp.float32),
        ],
        compiler_params=pltpu.CompilerParams(
            use_tc_tiling_on_sc=False, needs_layout_passes=False),
    )
    def kern(x_hbm, o_hbm, data_v, hist_v, histb_v, g_v, out_v):
        wid = lax.axis_index("s") * nc + lax.axis_index("c")

        def task(t, _):
            tid = wid * tasks_per_w + t
            bb = tid // DGROUPS
            dg = tid % DGROUPS
            _task_body(x_hbm, o_hbm, data_v, hist_v, histb_v, g_v, out_v,
                       bb, dg)
            return 0

        lax.fori_loop(0, tasks_per_w, task, 0)

    return kern(x)


def kernel(masked_layer):
    return _sc_topk(masked_layer)


# speedup vs baseline: 11.5510x; 1.0602x over previous
import functools

import jax
import jax.numpy as jnp
from jax import lax
from jax.experimental import pallas as pl
from jax.experimental.pallas import tpu as pltpu
from jax.experimental.pallas import tpu_sc as plsc

K = 64
B = 32
N = 4096
D = 128
LANES = 16
DGROUPS = D // LANES
NUM_TASKS = B * DGROUPS
NB1 = 1024
NBB = 256
NBC = 128


def _to_key(x):
    i = plsc.bitcast(x, jnp.int32)
    return jnp.where(i >= 0, i, i ^ jnp.int32(0x7FFFFFFF))


def _from_key(key):
    i = jnp.where(key >= 0, key, key ^ jnp.int32(0x7FFFFFFF))
    return plsc.bitcast(i, jnp.float32)


def _walk(hist_ref, start, acc0):
    zero16 = jnp.zeros((LANES,), jnp.int32)

    def scond(j):
        return (j >= 0) & (jnp.max(hist_ref[j]) == 0)

    j0 = lax.while_loop(scond, lambda j: j - 1, jnp.int32(start))

    def cond(carry):
        j, acc, _, _ = carry
        return (j >= 0) & (jnp.min(acc) < K)

    def body(carry):
        j, acc, bsel, above = carry
        h = hist_ref[j]
        nacc = acc + h
        newly = (acc < K) & (nacc >= K)
        bsel = jnp.where(newly, j, bsel)
        above = jnp.where(newly, acc, above)
        return j - 1, nacc, bsel, above

    _, _, bsel, above = lax.while_loop(cond, body, (j0, acc0, zero16, zero16))
    return bsel, above


def _clear(hist_ref, nbuckets):
    zero16 = jnp.zeros((LANES,), jnp.int32)

    @plsc.parallel_loop(0, nbuckets, unroll=8)
    def clr(j):
        hist_ref[j] = zero16


def _bitonic_sort_desc(g_ref):
    for k in range(6):
        for j in range(k, -1, -1):
            s = 1 << j
            half = 1 << k
            if k == 5:

                @plsc.parallel_loop(0, 32, unroll=4)
                def cxf(i, j=j, s=s):
                    p = ((i >> j) << (j + 1)) | (i & (s - 1))
                    q = p | s
                    a = g_ref[p]
                    b = g_ref[q]
                    g_ref[p] = jnp.maximum(a, b)
                    g_ref[q] = jnp.minimum(a, b)
            else:

                @plsc.parallel_loop(0, 16, unroll=4)
                def cxd(m, j=j, s=s, k=k, half=half):
                    i = ((m >> k) << (k + 1)) | (m & (half - 1))
                    p = ((i >> j) << (j + 1)) | (i & (s - 1))
                    q = p | s
                    a = g_ref[p]
                    b = g_ref[q]
                    g_ref[p] = jnp.maximum(a, b)
                    g_ref[q] = jnp.minimum(a, b)

                @plsc.parallel_loop(0, 16, unroll=4)
                def cxa(m, j=j, s=s, k=k, half=half):
                    i = (((m >> k) << (k + 1)) | (m & (half - 1))) | half
                    p = ((i >> j) << (j + 1)) | (i & (s - 1))
                    q = p | s
                    a = g_ref[p]
                    b = g_ref[q]
                    g_ref[p] = jnp.minimum(a, b)
                    g_ref[q] = jnp.maximum(a, b)


def _task_body(x_hbm, o_hbm, data_v, hist_v, histb_v, g_v, out_v, b, dg):
    lane = lax.iota(jnp.int32, LANES)
    ones = jnp.ones((LANES,), jnp.int32)
    zero16 = jnp.zeros((LANES,), jnp.int32)
    d0 = dg * LANES

    pltpu.sync_copy(x_hbm.at[b, :, pl.ds(d0, LANES)], data_v)

    _clear(hist_v, NB1)

    @plsc.parallel_loop(0, N, unroll=4)
    def p1(n):
        key = _to_key(data_v[n])
        data_v[n] = plsc.bitcast(key, jnp.float32)
        plsc.addupdate_scatter(hist_v, [(key >> 22) + 512, lane], ones)

    bsel1, c1 = _walk(hist_v, NB1 - 1, zero16)
    top1 = bsel1 - 512

    @plsc.parallel_loop(0, N, unroll=4, carry=zero16)
    def p2(n, cc):
        key = plsc.bitcast(data_v[n], jnp.int32)
        m = (key >> 22) >= top1
        plsc.store_scatter(data_v, [cc, lane], plsc.bitcast(key, jnp.float32),
                           mask=m)
        return cc + jnp.where(m, 1, 0)

    c_all = p2
    ncand = jnp.max(c_all)

    _clear(histb_v, NBB)

    @plsc.parallel_loop(0, ncand, unroll=2)
    def pb(n):
        key = plsc.bitcast(data_v[n], jnp.int32)
        m = (c_all > n) & ((key >> 22) == top1)
        plsc.addupdate_scatter(histb_v, [(key >> 14) & 0xFF, lane], ones,
                               mask=m)

    bselB, cB = _walk(histb_v, NBB - 1, c1)

    pre1B = (top1 << 8) | bselB
    _clear(histb_v, NBC)

    @plsc.parallel_loop(0, ncand, unroll=2)
    def pc(n):
        key = plsc.bitcast(data_v[n], jnp.int32)
        m = (c_all > n) & ((key >> 14) == pre1B)
        plsc.addupdate_scatter(histb_v, [(key >> 7) & 0x7F, lane], ones,
                               mask=m)

    bselC, cC = _walk(histb_v, NBC - 1, cB)

    pre1BC = (pre1B << 7) | bselC
    _clear(histb_v, NBC)

    @plsc.parallel_loop(0, ncand, unroll=2)
    def pd(n):
        key = plsc.bitcast(data_v[n], jnp.int32)
        m = (c_all > n) & ((key >> 7) == pre1BC)
        plsc.addupdate_scatter(histb_v, [key & 0x7F, lane], ones, mask=m)

    bselD, _ = _walk(histb_v, NBC - 1, cC)

    t_key = (pre1BC << 7) | bselD

    @plsc.parallel_loop(0, ncand, unroll=2, carry=zero16)
    def p4(n, ct):
        key = plsc.bitcast(data_v[n], jnp.int32)
        m = (c_all > n) & (key > t_key)
        plsc.store_scatter(g_v, [ct, lane], key, mask=m)
        return ct + jnp.where(m, 1, 0)

    total = p4

    @plsc.parallel_loop(0, K, unroll=4)
    def fill(kk):
        cur = g_v[kk]
        g_v[kk] = jnp.where(total <= kk, t_key, cur)

    _bitonic_sort_desc(g_v)

    lane64 = lane * K

    @plsc.parallel_loop(0, K, unroll=4)
    def emit(kk):
        x = _from_key(g_v[kk])
        plsc.store_scatter(out_v, [lane64 + kk], x)

    pltpu.sync_copy(out_v, o_hbm.at[b, pl.ds(dg * K * LANES, K * LANES)])


def _sc_topk(x):
    nc, ns = 2, 16
    nw = nc * ns
    tasks_per_w = NUM_TASKS // nw
    mesh = plsc.VectorSubcoreMesh(
        core_axis_name="c", subcore_axis_name="s", num_cores=nc, num_subcores=ns)

    @functools.partial(
        pl.kernel,
        out_type=jax.ShapeDtypeStruct((B, K * D), jnp.float32),
        mesh=mesh,
        scratch_types=[
            pltpu.VMEM((N, LANES), jnp.float32),
            pltpu.VMEM((NB1, LANES), jnp.int32),
            pltpu.VMEM((NBB, LANES), jnp.int32),
            pltpu.VMEM((K, LANES), jnp.int32),
            pltpu.VMEM((K * LANES,), jnp.float32),
        ],
        compiler_params=pltpu.CompilerParams(
            use_tc_tiling_on_sc=False, needs_layout_passes=False),
    )
    def kern(x_hbm, o_hbm, data_v, hist_v, histb_v, g_v, out_v):
        wid = lax.axis_index("s") * nc + lax.axis_index("c")

        def task(t, _):
            tid = wid * tasks_per_w + t
            bb = tid // DGROUPS
            dg = tid % DGROUPS
            _task_body(x_hbm, o_hbm, data_v, hist_v, histb_v, g_v, out_v,
                       bb, dg)
            return 0

        lax.fori_loop(0, tasks_per_w, task, 0)

    return kern(x)


def kernel(masked_layer):
    return _sc_topk(masked_layer)


# speedup vs baseline: 11.8117x; 1.0226x over previous
import functools

import jax
import jax.numpy as jnp
from jax import lax
from jax.experimental import pallas as pl
from jax.experimental.pallas import tpu as pltpu
from jax.experimental.pallas import tpu_sc as plsc

K = 64
B = 32
N = 4096
D = 128
LANES = 16
DGROUPS = D // LANES
NUM_TASKS = B * DGROUPS
NB1 = 1024
NBB = 256
NBC = 128


def _to_key(x):
    i = plsc.bitcast(x, jnp.int32)
    return jnp.where(i >= 0, i, i ^ jnp.int32(0x7FFFFFFF))


def _from_key(key):
    i = jnp.where(key >= 0, key, key ^ jnp.int32(0x7FFFFFFF))
    return plsc.bitcast(i, jnp.float32)


def _walk(hist_ref, start, acc0):
    zero16 = jnp.zeros((LANES,), jnp.int32)

    def scond(j):
        return (j >= 0) & (jnp.max(hist_ref[j]) == 0)

    j0 = lax.while_loop(scond, lambda j: j - 1, jnp.int32(start))

    def cond(carry):
        j, acc, _, _ = carry
        return (j >= 0) & (jnp.min(acc) < K)

    def body(carry):
        j, acc, bsel, above = carry
        h = hist_ref[j]
        nacc = acc + h
        newly = (acc < K) & (nacc >= K)
        bsel = jnp.where(newly, j, bsel)
        above = jnp.where(newly, acc, above)
        return j - 1, nacc, bsel, above

    _, _, bsel, above = lax.while_loop(cond, body, (j0, acc0, zero16, zero16))
    return bsel, above


def _clear(hist_ref, nbuckets):
    zero16 = jnp.zeros((LANES,), jnp.int32)

    @plsc.parallel_loop(0, nbuckets, unroll=8)
    def clr(j):
        hist_ref[j] = zero16


def _bitonic_sort_desc(g_ref):
    for k in range(6):
        for j in range(k, -1, -1):
            s = 1 << j
            half = 1 << k
            if k == 5:

                @plsc.parallel_loop(0, 32, unroll=4)
                def cxf(i, j=j, s=s):
                    p = ((i >> j) << (j + 1)) | (i & (s - 1))
                    q = p | s
                    a = g_ref[p]
                    b = g_ref[q]
                    g_ref[p] = jnp.maximum(a, b)
                    g_ref[q] = jnp.minimum(a, b)
            else:

                @plsc.parallel_loop(0, 16, unroll=4)
                def cxd(m, j=j, s=s, k=k, half=half):
                    i = ((m >> k) << (k + 1)) | (m & (half - 1))
                    p = ((i >> j) << (j + 1)) | (i & (s - 1))
                    q = p | s
                    a = g_ref[p]
                    b = g_ref[q]
                    g_ref[p] = jnp.maximum(a, b)
                    g_ref[q] = jnp.minimum(a, b)

                @plsc.parallel_loop(0, 16, unroll=4)
                def cxa(m, j=j, s=s, k=k, half=half):
                    i = (((m >> k) << (k + 1)) | (m & (half - 1))) | half
                    p = ((i >> j) << (j + 1)) | (i & (s - 1))
                    q = p | s
                    a = g_ref[p]
                    b = g_ref[q]
                    g_ref[p] = jnp.minimum(a, b)
                    g_ref[q] = jnp.maximum(a, b)


def _task_body(x_hbm, o_hbm, data_v, hist_v, histb_v, g_v, out_v, b, dg):
    lane = lax.iota(jnp.int32, LANES)
    ones = jnp.ones((LANES,), jnp.int32)
    zero16 = jnp.zeros((LANES,), jnp.int32)
    d0 = dg * LANES

    pltpu.sync_copy(x_hbm.at[b, :, pl.ds(d0, LANES)], data_v)

    _clear(hist_v, NB1)

    @plsc.parallel_loop(0, N, unroll=8)
    def p1(n):
        key = _to_key(data_v[n])
        data_v[n] = plsc.bitcast(key, jnp.float32)
        plsc.addupdate_scatter(hist_v, [(key >> 22) + 512, lane], ones)

    bsel1, c1 = _walk(hist_v, NB1 - 1, zero16)
    top1 = bsel1 - 512

    @plsc.parallel_loop(0, N, unroll=8, carry=zero16)
    def p2(n, cc):
        key = plsc.bitcast(data_v[n], jnp.int32)
        m = (key >> 22) >= top1
        plsc.store_scatter(data_v, [cc, lane], plsc.bitcast(key, jnp.float32),
                           mask=m)
        return cc + jnp.where(m, 1, 0)

    c_all = p2
    ncand = jnp.max(c_all)

    _clear(histb_v, NBB)

    @plsc.parallel_loop(0, ncand, unroll=2)
    def pb(n):
        key = plsc.bitcast(data_v[n], jnp.int32)
        m = (c_all > n) & ((key >> 22) == top1)
        plsc.addupdate_scatter(histb_v, [(key >> 14) & 0xFF, lane], ones,
                               mask=m)

    bselB, cB = _walk(histb_v, NBB - 1, c1)

    pre1B = (top1 << 8) | bselB
    _clear(histb_v, NBC)

    @plsc.parallel_loop(0, ncand, unroll=2)
    def pc(n):
        key = plsc.bitcast(data_v[n], jnp.int32)
        m = (c_all > n) & ((key >> 14) == pre1B)
        plsc.addupdate_scatter(histb_v, [(key >> 7) & 0x7F, lane], ones,
                               mask=m)

    bselC, cC = _walk(histb_v, NBC - 1, cB)

    pre1BC = (pre1B << 7) | bselC
    _clear(histb_v, NBC)

    @plsc.parallel_loop(0, ncand, unroll=2)
    def pd(n):
        key = plsc.bitcast(data_v[n], jnp.int32)
        m = (c_all > n) & ((key >> 7) == pre1BC)
        plsc.addupdate_scatter(histb_v, [key & 0x7F, lane], ones, mask=m)

    bselD, _ = _walk(histb_v, NBC - 1, cC)

    t_key = (pre1BC << 7) | bselD

    @plsc.parallel_loop(0, ncand, unroll=2, carry=zero16)
    def p4(n, ct):
        key = plsc.bitcast(data_v[n], jnp.int32)
        m = (c_all > n) & (key > t_key)
        plsc.store_scatter(g_v, [ct, lane], key, mask=m)
        return ct + jnp.where(m, 1, 0)

    total = p4

    @plsc.parallel_loop(0, K, unroll=4)
    def fill(kk):
        cur = g_v[kk]
        g_v[kk] = jnp.where(total <= kk, t_key, cur)

    _bitonic_sort_desc(g_v)

    lane64 = lane * K

    @plsc.parallel_loop(0, K, unroll=4)
    def emit(kk):
        x = _from_key(g_v[kk])
        plsc.store_scatter(out_v, [lane64 + kk], x)

    pltpu.sync_copy(out_v, o_hbm.at[b, pl.ds(dg * K * LANES, K * LANES)])


def _sc_topk(x):
    nc, ns = 2, 16
    nw = nc * ns
    tasks_per_w = NUM_TASKS // nw
    mesh = plsc.VectorSubcoreMesh(
        core_axis_name="c", subcore_axis_name="s", num_cores=nc, num_subcores=ns)

    @functools.partial(
        pl.kernel,
        out_type=jax.ShapeDtypeStruct((B, K * D), jnp.float32),
        mesh=mesh,
        scratch_types=[
            pltpu.VMEM((N, LANES), jnp.float32),
            pltpu.VMEM((NB1, LANES), jnp.int32),
            pltpu.VMEM((NBB, LANES), jnp.int32),
            pltpu.VMEM((K, LANES), jnp.int32),
            pltpu.VMEM((K * LANES,), jnp.float32),
        ],
        compiler_params=pltpu.CompilerParams(
            use_tc_tiling_on_sc=False, needs_layout_passes=False),
    )
    def kern(x_hbm, o_hbm, data_v, hist_v, histb_v, g_v, out_v):
        wid = lax.axis_index("s") * nc + lax.axis_index("c")

        def task(t, _):
            tid = wid * tasks_per_w + t
            bb = tid // DGROUPS
            dg = tid % DGROUPS
            _task_body(x_hbm, o_hbm, data_v, hist_v, histb_v, g_v, out_v,
                       bb, dg)
            return 0

        lax.fori_loop(0, tasks_per_w, task, 0)

    return kern(x)


def kernel(masked_layer):
    return _sc_topk(masked_layer)


# speedup vs baseline: 12.9563x; 1.0969x over previous
import functools

import jax
import jax.numpy as jnp
from jax import lax
from jax.experimental import pallas as pl
from jax.experimental.pallas import tpu as pltpu
from jax.experimental.pallas import tpu_sc as plsc

K = 64
B = 32
N = 4096
D = 128
LANES = 16
DGROUPS = D // LANES
NUM_TASKS = B * DGROUPS
NB1 = 1024
NBB = 256
NBC = 128
CAP = 512


def _to_key(x):
    i = plsc.bitcast(x, jnp.int32)
    return jnp.where(i >= 0, i, i ^ jnp.int32(0x7FFFFFFF))


def _from_key(key):
    i = jnp.where(key >= 0, key, key ^ jnp.int32(0x7FFFFFFF))
    return plsc.bitcast(i, jnp.float32)


def _walk(hist_ref, start, acc0):
    zero16 = jnp.zeros((LANES,), jnp.int32)

    def scond(j):
        return (j >= 0) & (jnp.max(hist_ref[j]) == 0)

    j0 = lax.while_loop(scond, lambda j: j - 1,
                        jnp.asarray(start, jnp.int32))

    def cond(carry):
        j, acc, _, _ = carry
        return (j >= 0) & (jnp.min(acc) < K)

    def body(carry):
        j, acc, bsel, above = carry
        h = hist_ref[j]
        nacc = acc + h
        newly = (acc < K) & (nacc >= K)
        bsel = jnp.where(newly, j, bsel)
        above = jnp.where(newly, acc, above)
        return j - 1, nacc, bsel, above

    _, _, bsel, above = lax.while_loop(cond, body, (j0, acc0, zero16, zero16))
    return bsel, above


def _clear(hist_ref, nbuckets):
    zero16 = jnp.zeros((LANES,), jnp.int32)

    @plsc.parallel_loop(0, nbuckets, unroll=8)
    def clr(j):
        hist_ref[j] = zero16


def _bitonic_sort_desc(g_ref):
    for k in range(6):
        for j in range(k, -1, -1):
            s = 1 << j
            half = 1 << k
            if k == 5:

                @plsc.parallel_loop(0, 32, unroll=4)
                def cxf(i, j=j, s=s):
                    p = ((i >> j) << (j + 1)) | (i & (s - 1))
                    q = p | s
                    a = g_ref[p]
                    b = g_ref[q]
                    g_ref[p] = jnp.maximum(a, b)
                    g_ref[q] = jnp.minimum(a, b)
            else:

                @plsc.parallel_loop(0, 16, unroll=4)
                def cxd(m, j=j, s=s, k=k, half=half):
                    i = ((m >> k) << (k + 1)) | (m & (half - 1))
                    p = ((i >> j) << (j + 1)) | (i & (s - 1))
                    q = p | s
                    a = g_ref[p]
                    b = g_ref[q]
                    g_ref[p] = jnp.maximum(a, b)
                    g_ref[q] = jnp.minimum(a, b)

                @plsc.parallel_loop(0, 16, unroll=4)
                def cxa(m, j=j, s=s, k=k, half=half):
                    i = (((m >> k) << (k + 1)) | (m & (half - 1))) | half
                    p = ((i >> j) << (j + 1)) | (i & (s - 1))
                    q = p | s
                    a = g_ref[p]
                    b = g_ref[q]
                    g_ref[p] = jnp.minimum(a, b)
                    g_ref[q] = jnp.maximum(a, b)


def _in_slice(x_hbm, tid):
    bb = tid // DGROUPS
    dg = tid % DGROUPS
    return x_hbm.at[bb, :, pl.ds(dg * LANES, LANES)], bb, dg


def _task_body(x_hbm, o_hbm, data_v, hist_v, histb_v, cand_v, g_v, out_v,
               sem, tid, do_prefetch):
    lane = lax.iota(jnp.int32, LANES)
    ones = jnp.ones((LANES,), jnp.int32)
    zero16 = jnp.zeros((LANES,), jnp.int32)
    src, bb, dg = _in_slice(x_hbm, tid)

    pltpu.make_async_copy(src, data_v, sem).wait()

    _clear(hist_v, NB1)

    @plsc.parallel_loop(0, N, unroll=8)
    def p1(n):
        key = _to_key(data_v[n])
        data_v[n] = plsc.bitcast(key, jnp.float32)
        plsc.addupdate_scatter(hist_v, [(key >> 22) + 512, lane], ones)

    bsel1, c1 = _walk(hist_v, NB1 - 1, zero16)
    top1 = bsel1 - 512

    @plsc.parallel_loop(0, N, unroll=8, carry=zero16)
    def p2(n, cc):
        key = plsc.bitcast(data_v[n], jnp.int32)
        m = (key >> 22) >= top1
        plsc.store_scatter(cand_v, [cc & (CAP - 1), lane],
                           plsc.bitcast(key, jnp.float32), mask=m)
        return cc + jnp.where(m, 1, 0)

    c_all = p2
    cmax = jnp.max(c_all)
    any_ov = cmax >= CAP
    ov = c_all >= CAP
    ncand = lax.min(cmax, jnp.int32(CAP))

    @pl.when(do_prefetch & jnp.logical_not(any_ov))
    def _prefetch_now():
        nsrc, _, _ = _in_slice(x_hbm, tid + 1)
        pltpu.async_copy(nsrc, data_v, sem)

    _clear(histb_v, NBB)

    @plsc.parallel_loop(0, ncand, unroll=2)
    def pb(n):
        key = plsc.bitcast(cand_v[n], jnp.int32)
        m = (c_all > n) & ((key >> 22) == top1)
        plsc.addupdate_scatter(histb_v, [(key >> 14) & 0xFF, lane], ones,
                               mask=m)

    bselB, cB = _walk(histb_v, NBB - 1, c1)

    pre1B = (top1 << 8) | bselB
    _clear(histb_v, NBC)

    @plsc.parallel_loop(0, ncand, unroll=2)
    def pc(n):
        key = plsc.bitcast(cand_v[n], jnp.int32)
        m = (c_all > n) & ((key >> 14) == pre1B)
        plsc.addupdate_scatter(histb_v, [(key >> 7) & 0x7F, lane], ones,
                               mask=m)

    bselC, cC = _walk(histb_v, NBC - 1, cB)

    pre1BC = (pre1B << 7) | bselC
    _clear(histb_v, NBC)

    @plsc.parallel_loop(0, ncand, unroll=2)
    def pd(n):
        key = plsc.bitcast(cand_v[n], jnp.int32)
        m = (c_all > n) & ((key >> 7) == pre1BC)
        plsc.addupdate_scatter(histb_v, [key & 0x7F, lane], ones, mask=m)

    bselD, _ = _walk(histb_v, NBC - 1, cC)
    t_key_n = (pre1BC << 7) | bselD

    nf = jnp.where(any_ov, N, 0)
    nfb = jnp.where(any_ov, NBB, 0)
    nfc = jnp.where(any_ov, NBC, 0)
    sfb = jnp.where(any_ov, NBB - 1, -1)
    sfc = jnp.where(any_ov, NBC - 1, -1)
    _clear(histb_v, nfb)

    @plsc.parallel_loop(0, nf, unroll=2)
    def pfb(n):
        key = plsc.bitcast(data_v[n], jnp.int32)
        m = ov & ((key >> 22) == top1)
        plsc.addupdate_scatter(histb_v, [(key >> 14) & 0xFF, lane], ones,
                               mask=m)

    bselB_f, cB_f = _walk(histb_v, sfb, c1)
    pre1B_f = (top1 << 8) | bselB_f
    _clear(histb_v, nfc)

    @plsc.parallel_loop(0, nf, unroll=2)
    def pfc(n):
        key = plsc.bitcast(data_v[n], jnp.int32)
        m = ov & ((key >> 14) == pre1B_f)
        plsc.addupdate_scatter(histb_v, [(key >> 7) & 0x7F, lane], ones,
                               mask=m)

    bselC_f, cC_f = _walk(histb_v, sfc, cB_f)
    pre1BC_f = (pre1B_f << 7) | bselC_f
    _clear(histb_v, nfc)

    @plsc.parallel_loop(0, nf, unroll=2)
    def pfd(n):
        key = plsc.bitcast(data_v[n], jnp.int32)
        m = ov & ((key >> 7) == pre1BC_f)
        plsc.addupdate_scatter(histb_v, [key & 0x7F, lane], ones, mask=m)

    bselD_f, _ = _walk(histb_v, sfc, cC_f)
    t_key = jnp.where(ov, (pre1BC_f << 7) | bselD_f, t_key_n)

    @plsc.parallel_loop(0, ncand, unroll=2, carry=zero16)
    def p4(n, ct):
        key = plsc.bitcast(cand_v[n], jnp.int32)
        m = (c_all < CAP) & (c_all > n) & (key > t_key)
        plsc.store_scatter(g_v, [ct, lane], key, mask=m)
        return ct + jnp.where(m, 1, 0)

    @plsc.parallel_loop(0, nf, unroll=2, carry=p4)
    def p4f(n, ct):
        key = plsc.bitcast(data_v[n], jnp.int32)
        m = ov & (key > t_key)
        plsc.store_scatter(g_v, [ct, lane], key, mask=m)
        return ct + jnp.where(m, 1, 0)

    total = p4f

    @pl.when(do_prefetch & any_ov)
    def _prefetch_late():
        nsrc, _, _ = _in_slice(x_hbm, tid + 1)
        pltpu.async_copy(nsrc, data_v, sem)

    @plsc.parallel_loop(0, K, unroll=4)
    def fill(kk):
        cur = g_v[kk]
        g_v[kk] = jnp.where(total <= kk, t_key, cur)

    _bitonic_sort_desc(g_v)

    lane64 = lane * K

    @plsc.parallel_loop(0, K, unroll=4)
    def emit(kk):
        x = _from_key(g_v[kk])
        plsc.store_scatter(out_v, [lane64 + kk], x)

    pltpu.sync_copy(out_v, o_hbm.at[bb, pl.ds(dg * K * LANES, K * LANES)])


def _sc_topk(x):
    nc, ns = 2, 16
    nw = nc * ns
    tasks_per_w = NUM_TASKS // nw
    mesh = plsc.VectorSubcoreMesh(
        core_axis_name="c", subcore_axis_name="s", num_cores=nc, num_subcores=ns)

    @functools.partial(
        pl.kernel,
        out_type=jax.ShapeDtypeStruct((B, K * D), jnp.float32),
        mesh=mesh,
        scratch_types=[
            pltpu.VMEM((N, LANES), jnp.float32),
            pltpu.VMEM((NB1, LANES), jnp.int32),
            pltpu.VMEM((NBB, LANES), jnp.int32),
            pltpu.VMEM((CAP, LANES), jnp.float32),
            pltpu.VMEM((K, LANES), jnp.int32),
            pltpu.VMEM((K * LANES,), jnp.float32),
            pltpu.SemaphoreType.DMA,
        ],
        compiler_params=pltpu.CompilerParams(
            use_tc_tiling_on_sc=False, needs_layout_passes=False),
    )
    def kern(x_hbm, o_hbm, data_v, hist_v, histb_v, cand_v, g_v, out_v, sem):
        wid = lax.axis_index("s") * nc + lax.axis_index("c")
        t0 = wid * tasks_per_w

        src0, _, _ = _in_slice(x_hbm, t0)
        pltpu.async_copy(src0, data_v, sem)

        def task(t, _):
            _task_body(x_hbm, o_hbm, data_v, hist_v, histb_v, cand_v, g_v,
                       out_v, sem, t0 + t, t < tasks_per_w - 1)
            return 0

        lax.fori_loop(0, tasks_per_w, task, 0)

    return kern(x)


def kernel(masked_layer):
    return _sc_topk(masked_layer)
